# baseline (speedup 1.0000x reference)
import jax
import jax.numpy as jnp
from jax import lax
from jax.experimental import pallas as pl
from jax.experimental.pallas import tpu as pltpu
from jax.experimental.pallas import tpu_sc as plsc

N = 10000
E = 320000
D = 128
NC = 2
NS = 16
K = 128
CHUNKS = E // K
CHC = CHUNKS // NC
FULL = CHC // NS
RPT = 624
REM = N - NS * RPT
NPAD = 10240
QPT = NPAD // NS


def _seg_sum_kernel(with_count: bool):
    mesh = plsc.VectorSubcoreMesh(core_axis_name="c", subcore_axis_name="s")
    out_type = [jax.ShapeDtypeStruct((NC, N, D), jnp.float32)]
    scratch = [
        pltpu.VMEM_SHARED((N, D), jnp.float32),
    ]
    scratch += [pltpu.VMEM((K,), jnp.int32)] * 4
    scratch += [pltpu.VMEM((1, K), jnp.int32)] * 4
    scratch += [pltpu.VMEM((K, D), jnp.float32)] * 2
    scratch += [pltpu.SemaphoreType.DMA] * 6
    if with_count:
        out_type.append(jax.ShapeDtypeStruct((NC, NPAD), jnp.float32))
        scratch += [
            pltpu.VMEM_SHARED((NPAD,), jnp.float32),
            pltpu.VMEM((K,), jnp.float32),
            pltpu.VMEM((QPT,), jnp.float32),
        ]

    def body(feat, edges, zf, *rest):
        if with_count:
            (zc, ones_h, out, cnt_out, acc, *pipe, cacc, ones_v, cbuf) = rest
        else:
            (out, acc, *pipe) = rest
        srcs = pipe[0:4]
        dsts = pipe[4:8]
        rows = pipe[8:10]
        isems = pipe[10:14]
        gsems = pipe[14:16]
        rows0 = rows[0]
        c = lax.axis_index("c")
        w = lax.axis_index("s")
        r0 = w * RPT
        ebase = c * (CHC * K) + w * K
        pltpu.sync_copy(zf.at[pl.ds(0, K)], rows0)
        for j in range(RPT // K):
            pltpu.sync_copy(rows0, acc.at[pl.ds(r0 + j * K, K)])
        tail = RPT % K
        pltpu.sync_copy(rows0.at[pl.ds(0, tail)],
                        acc.at[pl.ds(r0 + RPT - tail, tail)])

        @pl.when(w == 0)
        def _():
            pltpu.sync_copy(rows0.at[pl.ds(0, REM)],
                            acc.at[pl.ds(NS * RPT, REM)])
        if with_count:
            q0 = w * QPT
            pltpu.sync_copy(ones_h, ones_v)
            pltpu.sync_copy(zc.at[pl.ds(q0, QPT)], cbuf)
            pltpu.sync_copy(cbuf, cacc.at[pl.ds(q0, QPT)])
        plsc.subcore_barrier()

        def iload(k, j):
            off = ebase + k * (NS * K)
            pltpu.async_copy(edges.at[0, pl.ds(off, K)], srcs[j], isems[j])
            pltpu.async_copy(edges.at[1, pl.ds(off, K)], dsts[j].at[0],
                             isems[j])

        def iwait(k, j):
            off = ebase + k * (NS * K)
            pltpu.make_async_copy(edges.at[0, pl.ds(off, K)],
                                  srcs[j], isems[j]).wait()
            pltpu.make_async_copy(edges.at[1, pl.ds(off, K)],
                                  dsts[j].at[0], isems[j]).wait()

        def gissue(j, r):
            pltpu.async_copy(feat.at[srcs[j]], rows[r], gsems[r])

        def gwait(j, r):
            pltpu.make_async_copy(feat.at[srcs[j]], rows[r], gsems[r]).wait()

        def do_scatter(j, r):
            didx = dsts[j].at[0]
            pltpu.sync_copy(rows[r], acc.at[didx], add=True)
            if with_count:
                pltpu.sync_copy(ones_v, cacc.at[didx], add=True)

        extra = w < (CHC - NS * FULL)

        def valid(k):
            return (k < FULL) | ((k == FULL) & extra)

        for j in range(4):
            iload(j, j)
        iwait(0, 0)
        gissue(0, 0)
        iwait(1, 1)
        gissue(1, 1)

        def substep(k, j, r, in_loop):
            gwait(j, r)
            do_scatter(j, r)
            if in_loop:
                @pl.when(valid(k + 4))
                def _():
                    iload(k + 4, j)

                iwait(k + 2, (j + 2) % 4)
                gissue((j + 2) % 4, r)
            else:
                @pl.when(valid(k + 2))
                def _():
                    iwait(k + 2, (j + 2) % 4)
                    gissue((j + 2) % 4, r)

        def step(k4, carry):
            k = 4 * k4
            for j in range(4):
                substep(k + j, j, j % 2, in_loop=True)
            return carry

        nloop = (FULL - 2) // 4
        lax.fori_loop(0, nloop, step, 0)
        substep(4 * nloop, 0, 0, in_loop=False)
        substep(4 * nloop + 1, 1, 1, in_loop=False)

        @pl.when(extra)
        def _():
            gwait(2, 0)
            do_scatter(2, 0)
        plsc.subcore_barrier()
        for j in range(RPT // K):
            pltpu.sync_copy(acc.at[pl.ds(r0 + j * K, K)], rows0)
            pltpu.sync_copy(rows0, out.at[c, pl.ds(r0 + j * K, K)])
        pltpu.sync_copy(acc.at[pl.ds(r0 + RPT - tail, tail)],
                        rows0.at[pl.ds(0, tail)])
        pltpu.sync_copy(rows0.at[pl.ds(0, tail)],
                        out.at[c, pl.ds(r0 + RPT - tail, tail)])

        @pl.when(w == 0)
        def _():
            pltpu.sync_copy(acc.at[pl.ds(NS * RPT, REM)],
                            rows0.at[pl.ds(0, REM)])
            pltpu.sync_copy(rows0.at[pl.ds(0, REM)],
                            out.at[c, pl.ds(NS * RPT, REM)])
        if with_count:
            pltpu.sync_copy(cacc.at[pl.ds(q0, QPT)], cbuf)
            pltpu.sync_copy(cbuf, cnt_out.at[c, pl.ds(q0, QPT)])

    out = out_type if with_count else out_type[0]
    return pl.kernel(body, out_type=out, mesh=mesh, scratch_types=scratch)


_seg_sum_cnt = _seg_sum_kernel(with_count=True)
_seg_sum = _seg_sum_kernel(with_count=False)

_BN = 1000


def _lin_kernel():
    def body(x_ref, w_ref, b_ref, o_ref):
        o_ref[...] = jnp.dot(x_ref[...], w_ref[...],
                             preferred_element_type=jnp.float32) + b_ref[...]

    return pl.pallas_call(
        body,
        grid=(N // _BN,),
        in_specs=[
            pl.BlockSpec((_BN, D), lambda i: (i, 0)),
            pl.BlockSpec((D, D), lambda i: (0, 0)),
            pl.BlockSpec((1, D), lambda i: (0, 0)),
        ],
        out_specs=pl.BlockSpec((_BN, D), lambda i: (i, 0)),
        out_shape=jax.ShapeDtypeStruct((N, D), jnp.float32),
    )


def _sage_combine(relu: bool):
    def body(parts_ref, cnt_ref, r_ref, wl_ref, o_ref):
        s = parts_ref[0] + parts_ref[1]
        cnt1 = cnt_ref[0] + cnt_ref[1]
        mean = s / jnp.maximum(cnt1, 1.0)
        acc = jnp.dot(mean, wl_ref[...], preferred_element_type=jnp.float32)
        acc = acc + r_ref[...]
        o_ref[...] = jnp.maximum(acc, 0.0) if relu else acc

    return pl.pallas_call(
        body,
        grid=(N // _BN,),
        in_specs=[
            pl.BlockSpec((NC, _BN, D), lambda i: (0, i, 0)),
            pl.BlockSpec((NC, _BN, 1), lambda i: (0, i, 0)),
            pl.BlockSpec((_BN, D), lambda i: (i, 0)),
            pl.BlockSpec((D, D), lambda i: (0, 0)),
        ],
        out_specs=pl.BlockSpec((_BN, D), lambda i: (i, 0)),
        out_shape=jax.ShapeDtypeStruct((N, D), jnp.float32),
    )


_lin = _lin_kernel()
_combine_relu = _sage_combine(relu=True)
_combine_lin = _sage_combine(relu=False)


def kernel(x, edge_index, W1l, b1, W1r, W2l, b2, W2r):
    zf = jnp.zeros((N, D), jnp.float32)
    zc = jnp.zeros((NPAD,), jnp.float32)
    ones = jnp.ones((K,), jnp.float32)
    r1 = _lin(x, W1r, b1.reshape(1, D))
    parts1, cnt_p = _seg_sum_cnt(x, edge_index, zf, zc, ones)
    cnts = cnt_p[:, :N, None]
    h = _combine_relu(parts1, cnts, r1, W1l)
    r2 = _lin(h, W2r, b2.reshape(1, D))
    parts2 = _seg_sum(h, edge_index, zf)
    return _combine_lin(parts2, cnts, r2, W2l)


# speedup vs baseline: 1.0124x; 1.0124x over previous
import jax
import jax.numpy as jnp
from jax import lax
from jax.experimental import pallas as pl
from jax.experimental.pallas import tpu as pltpu
from jax.experimental.pallas import tpu_sc as plsc

N = 10000
E = 320000
D = 128
NC = 2
NS = 16
K = 128
CHUNKS = E // K
CHC = CHUNKS // NC
FULL = CHC // NS
RPT = 624
REM = N - NS * RPT
NPAD = 10240
QPT = NPAD // NS


def _seg_sum_kernel(with_count: bool):
    mesh = plsc.VectorSubcoreMesh(core_axis_name="c", subcore_axis_name="s")
    out_type = [jax.ShapeDtypeStruct((NC, N, D), jnp.float32)]
    scratch = [
        pltpu.VMEM_SHARED((N, D), jnp.float32),
    ]
    scratch += [pltpu.VMEM((K,), jnp.int32)] * 4
    scratch += [pltpu.VMEM((1, K), jnp.int32)] * 4
    scratch += [pltpu.VMEM((K, D), jnp.float32)] * 2
    scratch += [pltpu.SemaphoreType.DMA] * 6
    if with_count:
        out_type.append(jax.ShapeDtypeStruct((NC, NPAD), jnp.float32))
        scratch += [
            pltpu.VMEM_SHARED((NPAD,), jnp.float32),
            pltpu.VMEM((K,), jnp.float32),
            pltpu.VMEM((QPT,), jnp.float32),
        ]

    def body(feat, edges, zf, *rest):
        if with_count:
            (zc, ones_h, out, cnt_out, acc, *pipe, cacc, ones_v, cbuf) = rest
        else:
            (out, acc, *pipe) = rest
        srcs = pipe[0:4]
        dsts = pipe[4:8]
        rows = pipe[8:10]
        isems = pipe[10:14]
        gsems = pipe[14:16]
        rows0 = rows[0]
        c = lax.axis_index("c")
        w = lax.axis_index("s")
        r0 = w * RPT
        ebase = c * (CHC * K) + w * K
        pltpu.sync_copy(zf.at[pl.ds(0, K)], rows0)
        for j in range(RPT // K):
            pltpu.sync_copy(rows0, acc.at[pl.ds(r0 + j * K, K)])
        tail = RPT % K
        pltpu.sync_copy(rows0.at[pl.ds(0, tail)],
                        acc.at[pl.ds(r0 + RPT - tail, tail)])

        @pl.when(w == 0)
        def _():
            pltpu.sync_copy(rows0.at[pl.ds(0, REM)],
                            acc.at[pl.ds(NS * RPT, REM)])
        if with_count:
            q0 = w * QPT
            pltpu.sync_copy(ones_h, ones_v)
            pltpu.sync_copy(zc.at[pl.ds(q0, QPT)], cbuf)
            pltpu.sync_copy(cbuf, cacc.at[pl.ds(q0, QPT)])
        plsc.subcore_barrier()

        def iload(k, j):
            off = ebase + k * (NS * K)
            pltpu.async_copy(edges.at[0, pl.ds(off, K)], srcs[j], isems[j])
            pltpu.async_copy(edges.at[1, pl.ds(off, K)], dsts[j].at[0],
                             isems[j])

        def iwait(k, j):
            off = ebase + k * (NS * K)
            pltpu.make_async_copy(edges.at[0, pl.ds(off, K)],
                                  srcs[j], isems[j]).wait()
            pltpu.make_async_copy(edges.at[1, pl.ds(off, K)],
                                  dsts[j].at[0], isems[j]).wait()

        def gissue(j, r):
            pltpu.async_copy(feat.at[srcs[j]], rows[r], gsems[r])

        def gwait(j, r):
            pltpu.make_async_copy(feat.at[srcs[j]], rows[r], gsems[r]).wait()

        def do_scatter(j, r):
            didx = dsts[j].at[0]
            pltpu.sync_copy(rows[r], acc.at[didx], add=True)
            if with_count:
                pltpu.sync_copy(ones_v, cacc.at[didx], add=True)

        extra = w < (CHC - NS * FULL)

        def valid(k):
            return (k < FULL) | ((k == FULL) & extra)

        for j in range(4):
            iload(j, j)
        iwait(0, 0)
        gissue(0, 0)
        iwait(1, 1)
        gissue(1, 1)

        def substep(k, j, r, in_loop):
            gwait(j, r)
            do_scatter(j, r)
            if in_loop:
                @pl.when(valid(k + 4))
                def _():
                    iload(k + 4, j)

                iwait(k + 2, (j + 2) % 4)
                gissue((j + 2) % 4, r)
            else:
                @pl.when(valid(k + 2))
                def _():
                    iwait(k + 2, (j + 2) % 4)
                    gissue((j + 2) % 4, r)

        def step(k4, carry):
            k = 4 * k4
            for j in range(4):
                substep(k + j, j, j % 2, in_loop=True)
            return carry

        nloop = (FULL - 2) // 4
        lax.fori_loop(0, nloop, step, 0)
        substep(4 * nloop, 0, 0, in_loop=False)
        substep(4 * nloop + 1, 1, 1, in_loop=False)

        @pl.when(extra)
        def _():
            gwait(2, 0)
            do_scatter(2, 0)
        plsc.subcore_barrier()
        for j in range(RPT // K):
            pltpu.sync_copy(acc.at[pl.ds(r0 + j * K, K)], rows0)
            pltpu.sync_copy(rows0, out.at[c, pl.ds(r0 + j * K, K)])
        pltpu.sync_copy(acc.at[pl.ds(r0 + RPT - tail, tail)],
                        rows0.at[pl.ds(0, tail)])
        pltpu.sync_copy(rows0.at[pl.ds(0, tail)],
                        out.at[c, pl.ds(r0 + RPT - tail, tail)])

        @pl.when(w == 0)
        def _():
            pltpu.sync_copy(acc.at[pl.ds(NS * RPT, REM)],
                            rows0.at[pl.ds(0, REM)])
            pltpu.sync_copy(rows0.at[pl.ds(0, REM)],
                            out.at[c, pl.ds(NS * RPT, REM)])
        if with_count:
            pltpu.sync_copy(cacc.at[pl.ds(q0, QPT)], cbuf)
            pltpu.sync_copy(cbuf, cnt_out.at[c, pl.ds(q0, QPT)])

    out = out_type if with_count else out_type[0]
    return pl.kernel(body, out_type=out, mesh=mesh, scratch_types=scratch)


_seg_sum_cnt = _seg_sum_kernel(with_count=True)
_seg_sum = _seg_sum_kernel(with_count=False)

_BN = 1000


def _sage_combine(relu: bool):
    def body(parts_ref, cnt_ref, x_ref, wl_ref, wr_ref, b_ref, o_ref):
        s = parts_ref[0] + parts_ref[1]
        cnt1 = cnt_ref[0] + cnt_ref[1]
        mean = s / jnp.maximum(cnt1, 1.0)
        acc = jnp.dot(mean, wl_ref[...], preferred_element_type=jnp.float32)
        acc = acc + jnp.dot(x_ref[...], wr_ref[...],
                            preferred_element_type=jnp.float32)
        acc = acc + b_ref[...]
        o_ref[...] = jnp.maximum(acc, 0.0) if relu else acc

    return pl.pallas_call(
        body,
        grid=(N // _BN,),
        in_specs=[
            pl.BlockSpec((NC, _BN, D), lambda i: (0, i, 0)),
            pl.BlockSpec((NC, _BN, 1), lambda i: (0, i, 0)),
            pl.BlockSpec((_BN, D), lambda i: (i, 0)),
            pl.BlockSpec((D, D), lambda i: (0, 0)),
            pl.BlockSpec((D, D), lambda i: (0, 0)),
            pl.BlockSpec((1, D), lambda i: (0, 0)),
        ],
        out_specs=pl.BlockSpec((_BN, D), lambda i: (i, 0)),
        out_shape=jax.ShapeDtypeStruct((N, D), jnp.float32),
    )


_combine_relu = _sage_combine(relu=True)
_combine_lin = _sage_combine(relu=False)


def kernel(x, edge_index, W1l, b1, W1r, W2l, b2, W2r):
    zf = jnp.zeros((N, D), jnp.float32)
    zc = jnp.zeros((NPAD,), jnp.float32)
    ones = jnp.ones((K,), jnp.float32)
    parts1, cnt_p = _seg_sum_cnt(x, edge_index, zf, zc, ones)
    cnts = cnt_p[:, :N, None]
    h = _combine_relu(parts1, cnts, x, W1l, W1r, b1.reshape(1, D))
    parts2 = _seg_sum(h, edge_index, zf)
    return _combine_lin(parts2, cnts, h, W2l, W2r, b2.reshape(1, D))
